# Initial kernel scaffold; baseline (speedup 1.0000x reference)
import functools

import jax
import jax.numpy as jnp
from jax import lax
from jax.experimental import pallas as pl
from jax.experimental.pallas import tpu as pltpu
from jax.experimental.pallas import tpu_sc as plsc

N = 10000
NPAD = 10240
C = 128
NC, NS = 2, 16
NT = NC * NS
RPS = NPAD // NS
BLK = 512
TRASH = N

_HI = lax.Precision.HIGHEST


def _row_mask(i):
    rows = lax.broadcasted_iota(jnp.int32, (BLK, 1), 0) + i * BLK
    return rows < N


def _sc_scatter(y, src, dst):
    d = y.shape[1]
    ept = src.shape[0] // NT
    nchunk = ept // C
    mesh = plsc.VectorSubcoreMesh(core_axis_name="c", subcore_axis_name="s")

    @functools.partial(
        pl.kernel,
        mesh=mesh,
        out_type=jax.ShapeDtypeStruct((NC, NPAD, d), jnp.float32),
        scratch_types=[
            pltpu.VMEM((C,), jnp.int32),
            pltpu.VMEM((C,), jnp.int32),
            pltpu.VMEM((C, d), jnp.float32),
            pltpu.VMEM_SHARED((NPAD, d), jnp.float32),
        ],
    )
    def k(y_hbm, src_hbm, dst_hbm, out_hbm, src_v, dst_v, rows_v, acc_sh):
        cid = lax.axis_index("c")
        sid = lax.axis_index("s")
        r0 = sid * RPS
        pltpu.sync_copy(y_hbm.at[pl.ds(r0, RPS)], acc_sh.at[pl.ds(r0, RPS)])
        plsc.subcore_barrier()
        base0 = (cid * NS + sid) * ept

        @pl.loop(0, nchunk)
        def _(j):
            base = base0 + j * C
            pltpu.sync_copy(src_hbm.at[pl.ds(base, C)], src_v)
            pltpu.sync_copy(dst_hbm.at[pl.ds(base, C)], dst_v)
            pltpu.sync_copy(y_hbm.at[src_v], rows_v)
            pltpu.sync_copy(rows_v, acc_sh.at[dst_v], add=True)

        plsc.subcore_barrier()
        pltpu.sync_copy(acc_sh.at[pl.ds(r0, RPS)],
                        out_hbm.at[cid].at[pl.ds(r0, RPS)])

    return k(y, src, dst)


def _sc_degree(zeros, dst, ones):
    ept = dst.shape[0] // NT
    nchunk = ept // C
    mesh = plsc.VectorSubcoreMesh(core_axis_name="c", subcore_axis_name="s")

    @functools.partial(
        pl.kernel,
        mesh=mesh,
        out_type=jax.ShapeDtypeStruct((NC, NPAD, 16), jnp.float32),
        scratch_types=[
            pltpu.VMEM((C,), jnp.int32),
            pltpu.VMEM((C, 16), jnp.float32),
            pltpu.VMEM_SHARED((NPAD, 16), jnp.float32),
        ],
    )
    def k(z_hbm, dst_hbm, ones_hbm, out_hbm, dst_v, ones_v, acc_sh):
        cid = lax.axis_index("c")
        sid = lax.axis_index("s")
        r0 = sid * RPS
        pltpu.sync_copy(z_hbm.at[pl.ds(r0, RPS)], acc_sh.at[pl.ds(r0, RPS)])
        pltpu.sync_copy(ones_hbm, ones_v)
        plsc.subcore_barrier()
        base0 = (cid * NS + sid) * ept

        @pl.loop(0, nchunk)
        def _(j):
            base = base0 + j * C
            pltpu.sync_copy(dst_hbm.at[pl.ds(base, C)], dst_v)
            pltpu.sync_copy(ones_v, acc_sh.at[dst_v], add=True)

        plsc.subcore_barrier()
        pltpu.sync_copy(acc_sh.at[pl.ds(r0, RPS)],
                        out_hbm.at[cid].at[pl.ds(r0, RPS)])

    return k(zeros, dst, ones)


def _tc_mm(x, w):
    n, kdim = x.shape
    dout = w.shape[1]

    def body(x_ref, w_ref, o_ref):
        o_ref[...] = jnp.dot(x_ref[...], w_ref[...], precision=_HI,
                             preferred_element_type=jnp.float32)

    return pl.pallas_call(
        body,
        grid=(n // BLK,),
        in_specs=[pl.BlockSpec((BLK, kdim), lambda i: (i, 0)),
                  pl.BlockSpec((kdim, dout), lambda i: (0, 0))],
        out_specs=pl.BlockSpec((BLK, dout), lambda i: (i, 0)),
        out_shape=jax.ShapeDtypeStruct((n, dout), jnp.float32),
    )(x, w)


def _tc_scale(xw, dega, degb):
    d = xw.shape[1]

    def body(xw_ref, da_ref, db_ref, y_ref, dinv_ref):
        i = pl.program_id(0)
        mask = _row_mask(i)
        deg = jnp.where(mask, da_ref[...] + db_ref[...] + 1.0, 1.0)
        dinv = lax.rsqrt(jnp.maximum(deg, 1e-12))
        dinv_ref[...] = dinv
        y_ref[...] = jnp.where(mask, xw_ref[...] * dinv, 0.0)

    return pl.pallas_call(
        body,
        grid=(NPAD // BLK,),
        in_specs=[pl.BlockSpec((BLK, d), lambda i: (i, 0)),
                  pl.BlockSpec((BLK, 1), lambda i: (i, 0)),
                  pl.BlockSpec((BLK, 1), lambda i: (i, 0))],
        out_specs=[pl.BlockSpec((BLK, d), lambda i: (i, 0)),
                   pl.BlockSpec((BLK, 1), lambda i: (i, 0))],
        out_shape=[jax.ShapeDtypeStruct((NPAD, d), jnp.float32),
                   jax.ShapeDtypeStruct((NPAD, 1), jnp.float32)],
    )(xw, dega, degb)


def _tc_layer(p0, p1, yprev, dinv, b, w):
    din = yprev.shape[1]
    dout = w.shape[1]

    def body(p0_ref, p1_ref, y_ref, dinv_ref, b_ref, w_ref, o_ref):
        i = pl.program_id(0)
        mask = _row_mask(i)
        z = p0_ref[...] + p1_ref[...] - y_ref[...]
        t = z * dinv_ref[...] + b_ref[...]
        h = jnp.where(t >= 0, t, 0.1 * t)
        o = jnp.dot(h, w_ref[...], precision=_HI,
                    preferred_element_type=jnp.float32) * dinv_ref[...]
        o_ref[...] = jnp.where(mask, o, 0.0)

    return pl.pallas_call(
        body,
        grid=(NPAD // BLK,),
        in_specs=[pl.BlockSpec((BLK, din), lambda i: (i, 0)),
                  pl.BlockSpec((BLK, din), lambda i: (i, 0)),
                  pl.BlockSpec((BLK, din), lambda i: (i, 0)),
                  pl.BlockSpec((BLK, 1), lambda i: (i, 0)),
                  pl.BlockSpec((1, din), lambda i: (0, 0)),
                  pl.BlockSpec((din, dout), lambda i: (0, 0))],
        out_specs=pl.BlockSpec((BLK, dout), lambda i: (i, 0)),
        out_shape=jax.ShapeDtypeStruct((NPAD, dout), jnp.float32),
    )(p0, p1, yprev, dinv, b, w)


def _tc_final(p0, p1, yprev, dinv, b):
    def body(p0_ref, p1_ref, y_ref, dinv_ref, b_ref, o_ref):
        z = p0_ref[...] + p1_ref[...] - y_ref[...]
        t = z[:, 0:2] * dinv_ref[...] + b_ref[...]
        m = jnp.max(t, axis=1, keepdims=True)
        e = jnp.exp(t - m)
        o_ref[...] = e / jnp.sum(e, axis=1, keepdims=True)

    return pl.pallas_call(
        body,
        grid=(NPAD // BLK,),
        in_specs=[pl.BlockSpec((BLK, 16), lambda i: (i, 0)),
                  pl.BlockSpec((BLK, 16), lambda i: (i, 0)),
                  pl.BlockSpec((BLK, 16), lambda i: (i, 0)),
                  pl.BlockSpec((BLK, 1), lambda i: (i, 0)),
                  pl.BlockSpec((1, 2), lambda i: (0, 0))],
        out_specs=pl.BlockSpec((BLK, 2), lambda i: (i, 0)),
        out_shape=jax.ShapeDtypeStruct((NPAD, 2), jnp.float32),
    )(p0, p1, yprev, dinv, b)


def kernel(x, edge_index, W1, b1, W2, b2, W3, b3, W4, b4):
    f32 = jnp.float32
    e = edge_index.shape[1]
    epad = -(-e // (NT * C)) * (NT * C)
    fill = jnp.full((epad - e,), TRASH, jnp.int32)
    src = jnp.concatenate([edge_index[0], fill])
    dst = jnp.concatenate([edge_index[1], fill])

    x_p = jnp.pad(x, ((0, NPAD - N), (0, 0)))
    W2p = jnp.pad(W2, ((0, 0), (0, 12)))
    b2p = jnp.pad(b2, (0, 12)).reshape(1, 112)
    W3p = jnp.pad(W3, ((0, 12), (0, 0)))
    W4p = jnp.pad(W4, ((0, 0), (0, 14)))

    zeros16 = jnp.zeros((NPAD, 16), f32)
    ones16 = jnp.ones((C, 16), f32)

    xw1 = _tc_mm(x_p, W1)
    dp = _sc_degree(zeros16, dst, ones16)
    y1, dinv = _tc_scale(xw1, dp[0, :, 0:1], dp[1, :, 0:1])

    p = _sc_scatter(y1, src, dst)
    y2 = _tc_layer(p[0], p[1], y1, dinv, b1.reshape(1, 128), W2p)
    p = _sc_scatter(y2, src, dst)
    y3 = _tc_layer(p[0], p[1], y2, dinv, b2p, W3p)
    p = _sc_scatter(y3, src, dst)
    y4 = _tc_layer(p[0], p[1], y3, dinv, b3.reshape(1, 32), W4p)
    p = _sc_scatter(y4, src, dst)
    out = _tc_final(p[0], p[1], y4, dinv, b4.reshape(1, 2))
    return out[:N]


# speedup vs baseline: 10.8986x; 10.8986x over previous
import functools

import jax
import jax.numpy as jnp
from jax import lax
from jax.experimental import pallas as pl
from jax.experimental.pallas import tpu as pltpu
from jax.experimental.pallas import tpu_sc as plsc

N = 10000
NPAD = 10240
C = 128
NC, NS = 2, 16
NT = NC * NS
RPS = NPAD // NS
BLK = 512
TRASH = N

_HI = lax.Precision.HIGHEST
_SC_PARAMS = pltpu.CompilerParams(use_tc_tiling_on_sc=False)


def _row_mask(i):
    rows = lax.broadcasted_iota(jnp.int32, (BLK, 1), 0) + i * BLK
    return rows < N


def _sc_scatter(y, src, dst):
    d = y.shape[1]
    ept = src.shape[0] // NT
    nchunk = ept // C
    mesh = plsc.VectorSubcoreMesh(core_axis_name="c", subcore_axis_name="s")

    @functools.partial(
        pl.kernel,
        mesh=mesh,
        out_type=jax.ShapeDtypeStruct((NC, NPAD, d), jnp.float32),
        scratch_types=[
            pltpu.VMEM((C,), jnp.int32),
            pltpu.VMEM((C,), jnp.int32),
            pltpu.VMEM((C, d), jnp.float32),
            pltpu.VMEM_SHARED((NPAD, d), jnp.float32),
        ],
        compiler_params=_SC_PARAMS,
    )
    def k(y_hbm, src_hbm, dst_hbm, out_hbm, src_v, dst_v, rows_v, acc_sh):
        cid = lax.axis_index("c")
        sid = lax.axis_index("s")
        r0 = sid * RPS
        pltpu.sync_copy(y_hbm.at[pl.ds(r0, RPS)], acc_sh.at[pl.ds(r0, RPS)])
        plsc.subcore_barrier()
        base0 = (cid * NS + sid) * ept

        @pl.loop(0, nchunk)
        def _(j):
            base = base0 + j * C
            pltpu.sync_copy(src_hbm.at[pl.ds(base, C)], src_v)
            pltpu.sync_copy(dst_hbm.at[pl.ds(base, C)], dst_v)
            pltpu.sync_copy(y_hbm.at[src_v], rows_v)
            pltpu.sync_copy(rows_v, acc_sh.at[dst_v], add=True)

        plsc.subcore_barrier()
        pltpu.sync_copy(acc_sh.at[pl.ds(r0, RPS)],
                        out_hbm.at[cid].at[pl.ds(r0, RPS)])

    return k(y, src, dst)


def _sc_degree(zeros, dst, ones):
    ept = dst.shape[0] // NT
    nchunk = ept // C
    mesh = plsc.VectorSubcoreMesh(core_axis_name="c", subcore_axis_name="s")

    @functools.partial(
        pl.kernel,
        mesh=mesh,
        out_type=jax.ShapeDtypeStruct((NC, NPAD, 16), jnp.float32),
        scratch_types=[
            pltpu.VMEM((C,), jnp.int32),
            pltpu.VMEM((C, 16), jnp.float32),
            pltpu.VMEM_SHARED((NPAD, 16), jnp.float32),
        ],
        compiler_params=_SC_PARAMS,
    )
    def k(z_hbm, dst_hbm, ones_hbm, out_hbm, dst_v, ones_v, acc_sh):
        cid = lax.axis_index("c")
        sid = lax.axis_index("s")
        r0 = sid * RPS
        pltpu.sync_copy(z_hbm.at[pl.ds(r0, RPS)], acc_sh.at[pl.ds(r0, RPS)])
        pltpu.sync_copy(ones_hbm, ones_v)
        plsc.subcore_barrier()
        base0 = (cid * NS + sid) * ept

        @pl.loop(0, nchunk)
        def _(j):
            base = base0 + j * C
            pltpu.sync_copy(dst_hbm.at[pl.ds(base, C)], dst_v)
            pltpu.sync_copy(ones_v, acc_sh.at[dst_v], add=True)

        plsc.subcore_barrier()
        pltpu.sync_copy(acc_sh.at[pl.ds(r0, RPS)],
                        out_hbm.at[cid].at[pl.ds(r0, RPS)])

    return k(zeros, dst, ones)


def _tc_mm(x, w):
    n, kdim = x.shape
    dout = w.shape[1]

    def body(x_ref, w_ref, o_ref):
        o_ref[...] = jnp.dot(x_ref[...], w_ref[...], precision=_HI,
                             preferred_element_type=jnp.float32)

    return pl.pallas_call(
        body,
        grid=(n // BLK,),
        in_specs=[pl.BlockSpec((BLK, kdim), lambda i: (i, 0)),
                  pl.BlockSpec((kdim, dout), lambda i: (0, 0))],
        out_specs=pl.BlockSpec((BLK, dout), lambda i: (i, 0)),
        out_shape=jax.ShapeDtypeStruct((n, dout), jnp.float32),
    )(x, w)


def _tc_scale(xw, dega, degb):
    d = xw.shape[1]

    def body(xw_ref, da_ref, db_ref, y_ref, dinv_ref):
        i = pl.program_id(0)
        mask = _row_mask(i)
        deg = jnp.where(mask, da_ref[...] + db_ref[...] + 1.0, 1.0)
        dinv = lax.rsqrt(jnp.maximum(deg, 1e-12))
        dinv_ref[...] = dinv
        y_ref[...] = jnp.where(mask, xw_ref[...] * dinv, 0.0)

    return pl.pallas_call(
        body,
        grid=(NPAD // BLK,),
        in_specs=[pl.BlockSpec((BLK, d), lambda i: (i, 0)),
                  pl.BlockSpec((BLK, 1), lambda i: (i, 0)),
                  pl.BlockSpec((BLK, 1), lambda i: (i, 0))],
        out_specs=[pl.BlockSpec((BLK, d), lambda i: (i, 0)),
                   pl.BlockSpec((BLK, 1), lambda i: (i, 0))],
        out_shape=[jax.ShapeDtypeStruct((NPAD, d), jnp.float32),
                   jax.ShapeDtypeStruct((NPAD, 1), jnp.float32)],
    )(xw, dega, degb)


def _tc_layer(p0, p1, yprev, dinv, b, w):
    din = yprev.shape[1]
    dout = w.shape[1]

    def body(p0_ref, p1_ref, y_ref, dinv_ref, b_ref, w_ref, o_ref):
        i = pl.program_id(0)
        mask = _row_mask(i)
        z = p0_ref[...] + p1_ref[...] - y_ref[...]
        t = z * dinv_ref[...] + b_ref[...]
        h = jnp.where(t >= 0, t, 0.1 * t)
        o = jnp.dot(h, w_ref[...], precision=_HI,
                    preferred_element_type=jnp.float32) * dinv_ref[...]
        o_ref[...] = jnp.where(mask, o, 0.0)

    return pl.pallas_call(
        body,
        grid=(NPAD // BLK,),
        in_specs=[pl.BlockSpec((BLK, din), lambda i: (i, 0)),
                  pl.BlockSpec((BLK, din), lambda i: (i, 0)),
                  pl.BlockSpec((BLK, din), lambda i: (i, 0)),
                  pl.BlockSpec((BLK, 1), lambda i: (i, 0)),
                  pl.BlockSpec((1, din), lambda i: (0, 0)),
                  pl.BlockSpec((din, dout), lambda i: (0, 0))],
        out_specs=pl.BlockSpec((BLK, dout), lambda i: (i, 0)),
        out_shape=jax.ShapeDtypeStruct((NPAD, dout), jnp.float32),
    )(p0, p1, yprev, dinv, b, w)


def _tc_final(p0, p1, yprev, dinv, b):
    def body(p0_ref, p1_ref, y_ref, dinv_ref, b_ref, o_ref):
        z = p0_ref[...] + p1_ref[...] - y_ref[...]
        t = z[:, 0:2] * dinv_ref[...] + b_ref[...]
        m = jnp.max(t, axis=1, keepdims=True)
        e = jnp.exp(t - m)
        o_ref[...] = e / jnp.sum(e, axis=1, keepdims=True)

    return pl.pallas_call(
        body,
        grid=(NPAD // BLK,),
        in_specs=[pl.BlockSpec((BLK, 16), lambda i: (i, 0)),
                  pl.BlockSpec((BLK, 16), lambda i: (i, 0)),
                  pl.BlockSpec((BLK, 16), lambda i: (i, 0)),
                  pl.BlockSpec((BLK, 1), lambda i: (i, 0)),
                  pl.BlockSpec((1, 2), lambda i: (0, 0))],
        out_specs=pl.BlockSpec((BLK, 2), lambda i: (i, 0)),
        out_shape=jax.ShapeDtypeStruct((NPAD, 2), jnp.float32),
    )(p0, p1, yprev, dinv, b)


def kernel(x, edge_index, W1, b1, W2, b2, W3, b3, W4, b4):
    f32 = jnp.float32
    e = edge_index.shape[1]
    epad = -(-e // (NT * C)) * (NT * C)
    fill = jnp.full((epad - e,), TRASH, jnp.int32)
    src = jnp.concatenate([edge_index[0], fill])
    dst = jnp.concatenate([edge_index[1], fill])

    x_p = jnp.pad(x, ((0, NPAD - N), (0, 0)))
    W2p = jnp.pad(W2, ((0, 0), (0, 12)))
    b2p = jnp.pad(b2, (0, 12)).reshape(1, 112)
    W3p = jnp.pad(W3, ((0, 12), (0, 0)))
    W4p = jnp.pad(W4, ((0, 0), (0, 14)))

    zeros16 = jnp.zeros((NPAD, 16), f32)
    ones16 = jnp.ones((C, 16), f32)

    xw1 = _tc_mm(x_p, W1)
    dp = _sc_degree(zeros16, dst, ones16)
    y1, dinv = _tc_scale(xw1, dp[0, :, 0:1], dp[1, :, 0:1])

    p = _sc_scatter(y1, src, dst)
    y2 = _tc_layer(p[0], p[1], y1, dinv, b1.reshape(1, 128), W2p)
    p = _sc_scatter(y2, src, dst)
    y3 = _tc_layer(p[0], p[1], y2, dinv, b2p, W3p)
    p = _sc_scatter(y3, src, dst)
    y4 = _tc_layer(p[0], p[1], y3, dinv, b3.reshape(1, 32), W4p)
    p = _sc_scatter(y4, src, dst)
    out = _tc_final(p[0], p[1], y4, dinv, b4.reshape(1, 2))
    return out[:N]
